# Initial kernel scaffold; baseline (speedup 1.0000x reference)
import jax
import jax.numpy as jnp
from jax.experimental import pallas as pl


def kernel(x, es, W_gnn, b_gnn, W_aff, b_aff):
    raise NotImplementedError("write your pallas kernel here")


# speedup vs baseline: 5.2377x; 5.2377x over previous
import functools

import jax
import jax.numpy as jnp
from jax import lax
from jax.experimental import pallas as pl
from jax.experimental.pallas import tpu as pltpu
from jax.experimental.pallas import tpu_sc as plsc

NC = 2
NS = 16
CHUNK = 128


def _sc_segment_sum(x, src3, dst3, z, n_acc):
    n, d = x.shape
    kch = src3.shape[1]
    rps = (n // NS) & ~7
    rem = n - NS * rps

    mesh = plsc.VectorSubcoreMesh(core_axis_name="c", subcore_axis_name="s")

    @functools.partial(
        pl.kernel,
        out_type=jax.ShapeDtypeStruct((NC, n, d), jnp.float32),
        mesh=mesh,
        scratch_types=[
            pltpu.VMEM((kch, CHUNK), jnp.int32),
            pltpu.VMEM((kch, CHUNK), jnp.int32),
            pltpu.VMEM((CHUNK, d), jnp.float32),
            pltpu.VMEM_SHARED((n_acc, d), jnp.float32),
        ],
    )
    def sc_kernel(x_hbm, src_hbm, dst_hbm, z_hbm, out_hbm,
                  src_v, dst_v, rows_v, g_sh):
        cid = lax.axis_index("c")
        sid = lax.axis_index("s")
        wid = cid * NS + sid
        pltpu.sync_copy(src_hbm.at[wid], src_v)
        pltpu.sync_copy(dst_hbm.at[wid], dst_v)
        pltpu.sync_copy(z_hbm.at[pl.ds(0, rps)], g_sh.at[pl.ds(sid * rps, rps)])

        @pl.when(sid == 0)
        def _zero_rem():
            pltpu.sync_copy(z_hbm.at[pl.ds(0, rem)], g_sh.at[pl.ds(NS * rps, rem)])

        plsc.subcore_barrier()

        def body(j, carry):
            pltpu.sync_copy(x_hbm.at[src_v.at[j]], rows_v)
            pltpu.sync_copy(rows_v, g_sh.at[dst_v.at[j]], add=True)
            return carry

        lax.fori_loop(0, kch, body, 0)
        plsc.subcore_barrier()
        pltpu.sync_copy(
            g_sh.at[pl.ds(sid * rps, rps)],
            out_hbm.at[cid, pl.ds(sid * rps, rps)])

        @pl.when(sid == 0)
        def _write_rem():
            pltpu.sync_copy(
                g_sh.at[pl.ds(NS * rps, rem)],
                out_hbm.at[cid, pl.ds(NS * rps, rem)])

    return sc_kernel(x, src3, dst3, z)


def _tc_affine(gp, x, w_gnn, w_aff, b2):
    n, d = x.shape
    rows = 2000
    grid = n // rows

    def body(g0, g1, xb, wg, wa, b, out):
        h = g0[0] + g1[0]
        out[...] = (
            jnp.dot(h, wg[...], preferred_element_type=jnp.float32)
            + jnp.dot(xb[...], wa[...], preferred_element_type=jnp.float32)
            + b[...]
        )

    return pl.pallas_call(
        body,
        grid=(grid,),
        in_specs=[
            pl.BlockSpec((1, rows, d), lambda i: (0, i, 0)),
            pl.BlockSpec((1, rows, d), lambda i: (1, i, 0)),
            pl.BlockSpec((rows, d), lambda i: (i, 0)),
            pl.BlockSpec((d, d), lambda i: (0, 0)),
            pl.BlockSpec((d, d), lambda i: (0, 0)),
            pl.BlockSpec((1, d), lambda i: (0, 0)),
        ],
        out_specs=pl.BlockSpec((rows, d), lambda i: (i, 0)),
        out_shape=jax.ShapeDtypeStruct((n, d), jnp.float32),
    )(gp, gp, x, w_gnn, w_aff, b2)


def kernel(x, es, W_gnn, b_gnn, W_aff, b_aff):
    n, d = x.shape
    e = es.shape[1]
    nw = NC * NS
    kch = -(-e // (nw * CHUNK))
    e_pad = nw * kch * CHUNK
    src = jnp.concatenate(
        [es[0], jnp.zeros((e_pad - e,), jnp.int32)]).reshape(nw, kch, CHUNK)
    dst = jnp.concatenate(
        [es[1], jnp.full((e_pad - e,), n, jnp.int32)]).reshape(nw, kch, CHUNK)
    z = jnp.zeros((n // NS, d), jnp.float32)
    gp = _sc_segment_sum(x, src, dst, z, n_acc=n + 8)
    b2 = (b_gnn + b_aff).reshape(1, d)
    return _tc_affine(gp, x, W_gnn, W_aff, b2)
